# baseline (speedup 1.0000x reference)
import jax
import jax.numpy as jnp
from jax import lax
from jax.experimental import pallas as pl
from jax.experimental.pallas import tpu as pltpu
from jax.experimental.pallas import tpu_sc as plsc

EMB = 16
BATCH = 16384

NUM_WORKERS = 32
B_PER_W = BATCH // NUM_WORKERS
GATHER_CHUNK = 128
N_CHUNKS = B_PER_W // GATHER_CHUNK
LANES = 16


def _mf_body(user_hbm, item_hbm, ut_hbm, it_hbm, out_hbm,
             uidx_v, iidx_v, ucols_v, icols_v, out_v, sem):
    wid = lax.axis_index("s") * 2 + lax.axis_index("c")
    base = wid * B_PER_W

    pltpu.sync_copy(user_hbm.at[pl.ds(base, B_PER_W)], uidx_v)
    pltpu.sync_copy(item_hbm.at[pl.ds(base, B_PER_W)], iidx_v)

    pending = []
    for e in range(EMB):
        fresh = []
        for c in range(N_CHUNKS):
            sl = pl.ds(c * GATHER_CHUNK, GATHER_CHUNK)
            fresh.append(pltpu.async_copy(
                ut_hbm.at[e].at[uidx_v.at[sl]], ucols_v.at[e].at[sl], sem))
            fresh.append(pltpu.async_copy(
                it_hbm.at[e].at[iidx_v.at[sl]], icols_v.at[e].at[sl], sem))
        for cp in pending:
            cp.wait()
        pending = fresh
    for cp in pending:
        cp.wait()

    def group_body(g, _):
        off = pl.multiple_of(g * LANES, LANES)
        acc = ucols_v[0, pl.ds(off, LANES)] * icols_v[0, pl.ds(off, LANES)]
        for e in range(1, EMB):
            acc = acc + (ucols_v[e, pl.ds(off, LANES)]
                         * icols_v[e, pl.ds(off, LANES)])
        out_v[pl.ds(off, LANES)] = acc
        return ()

    lax.fori_loop(0, B_PER_W // LANES, group_body, ())

    pltpu.sync_copy(out_v, out_hbm.at[pl.ds(base, B_PER_W)])


@jax.jit
def _mf(user, item, ut, it):
    mesh = plsc.VectorSubcoreMesh(core_axis_name="c", subcore_axis_name="s")
    f = pl.kernel(
        _mf_body,
        mesh=mesh,
        compiler_params=pltpu.CompilerParams(
            needs_layout_passes=False, use_tc_tiling_on_sc=False),
        out_type=jax.ShapeDtypeStruct((BATCH,), jnp.float32),
        scratch_types=[
            pltpu.VMEM((B_PER_W,), jnp.int32),
            pltpu.VMEM((B_PER_W,), jnp.int32),
            pltpu.VMEM((EMB, B_PER_W), jnp.float32),
            pltpu.VMEM((EMB, B_PER_W), jnp.float32),
            pltpu.VMEM((B_PER_W,), jnp.float32),
            pltpu.SemaphoreType.DMA,
        ],
    )
    return f(user, item, ut, it)


def kernel(user, item, embed_user_GMF, embed_item_GMF):
    user = user.astype(jnp.int32)
    item = item.astype(jnp.int32)
    return _mf(user, item, embed_user_GMF.T, embed_item_GMF.T)


# speedup vs baseline: 3.1985x; 3.1985x over previous
import jax
import jax.numpy as jnp
from jax import lax
from jax.experimental import pallas as pl
from jax.experimental.pallas import tpu as pltpu
from jax.experimental.pallas import tpu_sc as plsc

EMB = 16
BATCH = 16384

NUM_WORKERS = 32
B_PER_W = BATCH // NUM_WORKERS
GATHER_CHUNK = 128
N_CHUNKS = B_PER_W // GATHER_CHUNK
LANES = 16
N_GROUPS = B_PER_W // LANES


def _mf_body(user_hbm, item_hbm, utab_hbm, itab_hbm, out_hbm,
             uidx_v, iidx_v, urows_v, irows_v, out_v, sem):
    wid = lax.axis_index("s") * 2 + lax.axis_index("c")
    base = wid * B_PER_W

    pltpu.sync_copy(user_hbm.at[pl.ds(base, B_PER_W)], uidx_v)
    pltpu.sync_copy(item_hbm.at[pl.ds(base, B_PER_W)], iidx_v)

    copies = []
    for c in range(N_CHUNKS):
        sl = pl.ds(c * GATHER_CHUNK, GATHER_CHUNK)
        copies.append(pltpu.async_copy(
            utab_hbm.at[uidx_v.at[sl]], urows_v.at[sl], sem))
        copies.append(pltpu.async_copy(
            itab_hbm.at[iidx_v.at[sl]], irows_v.at[sl], sem))
    for cp in copies:
        cp.wait()

    lanes = lax.iota(jnp.int32, LANES)

    def group_body(g, _):
        rows = g * LANES + lanes
        acc = jnp.zeros((LANES,), jnp.float32)
        for e in range(EMB):
            cols = jnp.full((LANES,), e, jnp.int32)
            uv = plsc.load_gather(urows_v, [rows, cols])
            iv = plsc.load_gather(irows_v, [rows, cols])
            acc = acc + uv * iv
        out_v[pl.ds(g * LANES, LANES)] = acc
        return ()

    lax.fori_loop(0, N_GROUPS, group_body, ())

    pltpu.sync_copy(out_v, out_hbm.at[pl.ds(base, B_PER_W)])


@jax.jit
def _mf(user, item, embed_user_GMF, embed_item_GMF):
    mesh = plsc.VectorSubcoreMesh(core_axis_name="c", subcore_axis_name="s")
    f = pl.kernel(
        _mf_body,
        mesh=mesh,
        compiler_params=pltpu.CompilerParams(
            needs_layout_passes=False, use_tc_tiling_on_sc=False),
        out_type=jax.ShapeDtypeStruct((BATCH,), jnp.float32),
        scratch_types=[
            pltpu.VMEM((B_PER_W,), jnp.int32),
            pltpu.VMEM((B_PER_W,), jnp.int32),
            pltpu.VMEM((B_PER_W, EMB), jnp.float32),
            pltpu.VMEM((B_PER_W, EMB), jnp.float32),
            pltpu.VMEM((B_PER_W,), jnp.float32),
            pltpu.SemaphoreType.DMA,
        ],
    )
    return f(user, item, embed_user_GMF, embed_item_GMF)


def kernel(user, item, embed_user_GMF, embed_item_GMF):
    user = user.astype(jnp.int32)
    item = item.astype(jnp.int32)
    return _mf(user, item, embed_user_GMF, embed_item_GMF)
